# baseline (speedup 1.0000x reference)
import functools

import jax
import jax.numpy as jnp
from jax.experimental import pallas as pl
from jax.experimental.pallas import tpu as pltpu

EPS = 1e-5


def _square_distance(src, dst):
    d = -2.0 * jnp.einsum('bnc,bmc->bnm', src, dst)
    d = d + jnp.sum(src ** 2, -1)[:, :, None]
    d = d + jnp.sum(dst ** 2, -1)[:, None, :]
    return d


def _index_points(points, idx):
    return jax.vmap(lambda p, i: p[i])(points, idx)


def _farthest_point_sample(xyz, npoint):
    B, N, _ = xyz.shape
    def body(i, state):
        cent, dist, far = state
        cent = cent.at[:, i].set(far)
        c = jnp.take_along_axis(xyz, far[:, None, None], axis=1)
        d = jnp.sum((xyz - c) ** 2, -1)
        dist = jnp.minimum(dist, d)
        far = jnp.argmax(dist, axis=-1).astype(jnp.int32)
        return cent, dist, far
    cent = jnp.zeros((B, npoint), jnp.int32)
    dist = jnp.full((B, N), 1e10, jnp.float32)
    far = jnp.zeros((B,), jnp.int32)
    cent, _, _ = jax.lax.fori_loop(0, npoint, body, (cent, dist, far))
    return cent


def _knn_point(nsample, xyz, new_xyz):
    d = _square_distance(new_xyz, xyz)
    _, idx = jax.lax.top_k(-d, nsample)
    return idx


def _fps_knn_body(npoint, nsample, xyz_nc_ref, xyz_cn_ref,
                  new_xyz_ref, idx_ref, d_scratch):
    xyz_nc = xyz_nc_ref[0]
    xyz_cn = xyz_cn_ref[0]
    N = xyz_cn.shape[1]
    lane_iota = jax.lax.broadcasted_iota(jnp.int32, (1, N), 1)
    cent_iota = jax.lax.broadcasted_iota(jnp.int32, (npoint, 1), 0)

    def body(i, carry):
        cent, dist, far = carry
        cent = jnp.where(cent_iota == i, far, cent)
        sel = lane_iota == far
        c = jnp.sum(jnp.where(sel, xyz_cn, 0.0), axis=1, keepdims=True)
        d = jnp.sum((xyz_cn - c) ** 2, axis=0, keepdims=True)
        dist = jnp.minimum(dist, d)
        far = jnp.argmax(dist).astype(jnp.int32)
        return cent, dist, far

    cent0 = jnp.zeros((npoint, 1), jnp.int32)
    dist0 = jnp.full((1, N), 1e10, jnp.float32)
    cent, _, _ = jax.lax.fori_loop(0, npoint, body,
                                   (cent0, dist0, jnp.int32(0)))

    onehot = (cent == lane_iota).astype(jnp.float32)
    new_xyz = jnp.dot(onehot, xyz_nc,
                      precision=jax.lax.Precision.HIGHEST)
    new_xyz_ref[0] = new_xyz

    D = -2.0 * jnp.dot(new_xyz, xyz_cn)
    D = D + jnp.sum(new_xyz ** 2, axis=1, keepdims=True)
    D = D + jnp.sum(xyz_cn ** 2, axis=0, keepdims=True)
    d_scratch[...] = D

    samp_iota = jax.lax.broadcasted_iota(jnp.int32, (npoint, nsample), 1)

    def topk_body(k, idx_acc):
        Dk = d_scratch[...]
        amin = jnp.argmin(Dk, axis=1).astype(jnp.int32)[:, None]
        idx_acc = jnp.where(samp_iota == k, amin, idx_acc)
        d_scratch[...] = jnp.where(lane_iota == amin, jnp.float32(jnp.inf), Dk)
        return idx_acc

    idx_ref[0] = jax.lax.fori_loop(
        0, nsample, topk_body, jnp.zeros((npoint, nsample), jnp.int32))


def _fps_knn(xyz_nc, npoint, nsample):
    B, N, _ = xyz_nc.shape
    xyz_cn = jnp.transpose(xyz_nc, (0, 2, 1))
    body = functools.partial(_fps_knn_body, npoint, nsample)
    new_xyz, idx = pl.pallas_call(
        body,
        grid=(B,),
        in_specs=[
            pl.BlockSpec((1, N, 3), lambda b: (b, 0, 0)),
            pl.BlockSpec((1, 3, N), lambda b: (b, 0, 0)),
        ],
        out_specs=[
            pl.BlockSpec((1, npoint, 3), lambda b: (b, 0, 0)),
            pl.BlockSpec((1, npoint, nsample), lambda b: (b, 0, 0)),
        ],
        out_shape=[
            jax.ShapeDtypeStruct((B, npoint, 3), jnp.float32),
            jax.ShapeDtypeStruct((B, npoint, nsample), jnp.int32),
        ],
        scratch_shapes=[pltpu.VMEM((npoint, N), jnp.float32)],
    )(xyz_nc, xyz_cn)
    return new_xyz, idx


def _compute_density(xyz, bandwidth):
    sq = _square_distance(xyz, xyz)
    g = jnp.exp(-sq / (2.0 * bandwidth * bandwidth)) / (2.5 * bandwidth)
    return jnp.mean(g, axis=-1)


def _density_body(bw, rows_ref, xyz_cn_ref, out_ref):
    rows = rows_ref[0]
    xyz_cn = xyz_cn_ref[0]
    D = -2.0 * jnp.dot(rows, xyz_cn)
    D = D + jnp.sum(rows ** 2, axis=1, keepdims=True)
    D = D + jnp.sum(xyz_cn ** 2, axis=0, keepdims=True)
    g = jnp.exp(-D / (2.0 * bw * bw)) / (2.5 * bw)
    out_ref[0] = 1.0 / jnp.mean(g, axis=1, keepdims=True)


def _inv_density(xyz_nc, bandwidth):
    B, N, _ = xyz_nc.shape
    rb = min(N, 512)
    xyz_cn = jnp.transpose(xyz_nc, (0, 2, 1))
    body = functools.partial(_density_body, bandwidth)
    out = pl.pallas_call(
        body,
        grid=(B, N // rb),
        in_specs=[
            pl.BlockSpec((1, rb, 3), lambda b, r: (b, r, 0)),
            pl.BlockSpec((1, 3, N), lambda b, r: (b, 0, 0)),
        ],
        out_specs=pl.BlockSpec((1, rb, 1), lambda b, r: (b, r, 0)),
        out_shape=jax.ShapeDtypeStruct((B, N, 1), jnp.float32),
    )(xyz_nc, xyz_cn)
    return out[:, :, 0]


def _conv_chain(layers, x):
    for L in layers:
        x = jnp.einsum('bckm,oc->bokm', x, L['w']) + L['b'][None, :, None, None]
        m = jnp.mean(x, axis=(0, 2, 3), keepdims=True)
        v = jnp.var(x, axis=(0, 2, 3), keepdims=True)
        x = (x - m) / jnp.sqrt(v + EPS) * L['g'][None, :, None, None] + L['beta'][None, :, None, None]
        x = jax.nn.relu(x)
    return x


def _pointconv_sa(p, xyz, points, npoint, nsample, bandwidth, group_all):
    B = xyz.shape[0]
    N = xyz.shape[2]
    xyz_t = jnp.transpose(xyz, (0, 2, 1))
    pts_t = jnp.transpose(points, (0, 2, 1))
    inv_density = _inv_density(xyz_t, bandwidth)
    if group_all:
        new_xyz = jnp.zeros((B, 1, 3), jnp.float32)
        grouped_xyz_norm = xyz_t[:, None, :, :]
        new_points = jnp.concatenate([grouped_xyz_norm, pts_t[:, None, :, :]], axis=-1)
        grouped_density = inv_density.reshape(B, 1, N, 1)
        npt = 1
    else:
        new_xyz, idx = _fps_knn(xyz_t, npoint, nsample)
        grouped_xyz = _index_points(xyz_t, idx)
        grouped_xyz_norm = grouped_xyz - new_xyz[:, :, None, :]
        grouped_points = _index_points(pts_t, idx)
        new_points = jnp.concatenate([grouped_xyz_norm, grouped_points], axis=-1)
        grouped_density = _index_points(inv_density[:, :, None], idx)
        npt = npoint
    x = jnp.transpose(new_points, (0, 3, 2, 1))
    x = _conv_chain(p['mlp'], x)
    inv_max = jnp.max(grouped_density, axis=2, keepdims=True)
    dscale = grouped_density / inv_max
    dscale = jnp.transpose(dscale, (0, 3, 2, 1))
    dscale = _conv_chain(p['dn'], dscale)
    x = x * dscale
    gx = jnp.transpose(grouped_xyz_norm, (0, 3, 2, 1))
    w = _conv_chain(p['wn'], gx)
    xp = jnp.transpose(x, (0, 3, 1, 2))
    wp = jnp.transpose(w, (0, 3, 2, 1))
    out = jnp.matmul(xp, wp).reshape(B, npt, -1)
    out = out @ p['lin_w'].T + p['lin_b']
    out = jnp.transpose(out, (0, 2, 1))
    m = jnp.mean(out, axis=(0, 2), keepdims=True)
    v = jnp.var(out, axis=(0, 2), keepdims=True)
    out = (out - m) / jnp.sqrt(v + EPS) * p['bnl_g'][None, :, None] + p['bnl_b'][None, :, None]
    out = jax.nn.relu(out)
    return jnp.transpose(new_xyz, (0, 2, 1)), out


def _ln_elu(x, g, beta):
    m = jnp.mean(x, axis=-1, keepdims=True)
    v = jnp.mean((x - m) ** 2, axis=-1, keepdims=True)
    x = (x - m) / jnp.sqrt(v + EPS) * g + beta
    return jnp.where(x > 0, x, jnp.exp(jnp.minimum(x, 0.0)) - 1.0)


def _qhead_body(q_ref, xpc_ref,
                w1q, b1q, g1q, beta1q,
                w2q, b2q, g2q, beta2q,
                w3q, b3q, g3q, beta3q,
                w1, b1, g1, beta1,
                w2, b2, g2, beta2,
                w3, b3,
                o_ref):
    q = q_ref[...]
    x = jnp.dot(q, w1q[...].T) + b1q[...]
    x = _ln_elu(x, g1q[...], beta1q[...])
    x = jnp.dot(x, w2q[...].T) + b2q[...]
    x = _ln_elu(x, g2q[...], beta2q[...])
    x = jnp.dot(x, w3q[...].T) + b3q[...]
    x = _ln_elu(x, g3q[...], beta3q[...])
    w1v = w1[...]
    wa = w1v[:, :256]
    wb = w1v[:, 256:]
    xpc = xpc_ref[0]
    cb = jnp.dot(xpc, wa.T)
    h = jnp.dot(x, wb.T) + cb + b1[...]
    h = _ln_elu(h, g1[...], beta1[...])
    h = jnp.dot(h, w2[...].T) + b2[...]
    h = _ln_elu(h, g2[...], beta2[...])
    o_ref[...] = jnp.sum(h * w3[...], axis=-1, keepdims=True) + b3[0, 0]


def _query_head(x_pc, query, params):
    B, num_qrs, _ = query.shape
    tok = B * num_qrs
    blk = 1024
    blocks_per_batch = num_qrs // blk
    q = query.reshape(tok, 3)

    def wspec(shape):
        return pl.BlockSpec(shape, lambda i: (0,) * len(shape))

    lins = []
    for name in ('fc1q', 'fc2q', 'fc3q', 'fc1', 'fc2'):
        L = params[name]
        lins += [L['w'], L['b'], L['g'], L['beta']]
    lins += [params['fc3']['w'], params['fc3']['b'].reshape(1, 1)]

    in_specs = [
        pl.BlockSpec((blk, 3), lambda i: (i, 0)),
        pl.BlockSpec((1, 1, 256), lambda i: (i // blocks_per_batch, 0, 0)),
    ] + [wspec(w.shape) for w in lins]
    in_specs[-1] = pl.BlockSpec(memory_space=pltpu.SMEM)

    out = pl.pallas_call(
        _qhead_body,
        grid=(tok // blk,),
        in_specs=in_specs,
        out_specs=pl.BlockSpec((blk, 1), lambda i: (i, 0)),
        out_shape=jax.ShapeDtypeStruct((tok, 1), jnp.float32),
    )(q, x_pc[:, None, :], *lins)
    return out


def kernel(pc, query, params):
    B = pc.shape[0]
    l0_xyz = pc[:, :3, :]
    l1_xyz, l1_pts = _pointconv_sa(params['sa1'], l0_xyz, pc, 512, 32, 0.1, False)
    l2_xyz, l2_pts = _pointconv_sa(params['sa2'], l1_xyz, l1_pts, 128, 64, 0.2, False)
    l3_xyz, l3_pts = _pointconv_sa(params['sa3'], l2_xyz, l2_pts, 1, None, 0.4, True)
    x_pc = l3_pts.reshape(B, 256)
    return _query_head(x_pc, query, params)


# speedup vs baseline: 2.2614x; 2.2614x over previous
import functools

import jax
import jax.numpy as jnp
from jax.experimental import pallas as pl
from jax.experimental.pallas import tpu as pltpu

EPS = 1e-5


def _square_distance(src, dst):
    d = -2.0 * jnp.einsum('bnc,bmc->bnm', src, dst)
    d = d + jnp.sum(src ** 2, -1)[:, :, None]
    d = d + jnp.sum(dst ** 2, -1)[:, None, :]
    return d


def _index_points(points, idx):
    return jax.vmap(lambda p, i: p[i])(points, idx)


def _farthest_point_sample(xyz, npoint):
    B, N, _ = xyz.shape
    def body(i, state):
        cent, dist, far = state
        cent = cent.at[:, i].set(far)
        c = jnp.take_along_axis(xyz, far[:, None, None], axis=1)
        d = jnp.sum((xyz - c) ** 2, -1)
        dist = jnp.minimum(dist, d)
        far = jnp.argmax(dist, axis=-1).astype(jnp.int32)
        return cent, dist, far
    cent = jnp.zeros((B, npoint), jnp.int32)
    dist = jnp.full((B, N), 1e10, jnp.float32)
    far = jnp.zeros((B,), jnp.int32)
    cent, _, _ = jax.lax.fori_loop(0, npoint, body, (cent, dist, far))
    return cent


def _knn_point(nsample, xyz, new_xyz):
    d = _square_distance(new_xyz, xyz)
    _, idx = jax.lax.top_k(-d, nsample)
    return idx


def _fps_knn_body(npoint, nsample, xyz_nc_ref, xyz_cn_ref,
                  new_xyz_ref, idx_ref, d_scratch):
    xyz_nc = xyz_nc_ref[0]
    xyz_cn = xyz_cn_ref[0]
    N = xyz_cn.shape[1]
    lane_iota = jax.lax.broadcasted_iota(jnp.int32, (1, N), 1)
    cent_iota = jax.lax.broadcasted_iota(jnp.int32, (npoint, 1), 0)

    def body(i, carry):
        cent, dist, far = carry
        cent = jnp.where(cent_iota == i, far, cent)
        sel = lane_iota == far
        c = jnp.sum(jnp.where(sel, xyz_cn, 0.0), axis=1, keepdims=True)
        d = jnp.sum((xyz_cn - c) ** 2, axis=0, keepdims=True)
        dist = jnp.minimum(dist, d)
        far = jnp.argmax(dist).astype(jnp.int32)
        return cent, dist, far

    cent0 = jnp.zeros((npoint, 1), jnp.int32)
    dist0 = jnp.full((1, N), 1e10, jnp.float32)
    cent, _, _ = jax.lax.fori_loop(0, npoint, body,
                                   (cent0, dist0, jnp.int32(0)))

    onehot = (cent == lane_iota).astype(jnp.float32)
    new_xyz = jnp.dot(onehot, xyz_nc,
                      precision=jax.lax.Precision.HIGHEST)
    new_xyz_ref[0] = new_xyz

    D = -2.0 * jnp.dot(new_xyz, xyz_cn)
    D = D + jnp.sum(new_xyz ** 2, axis=1, keepdims=True)
    D = D + jnp.sum(xyz_cn ** 2, axis=0, keepdims=True)
    d_scratch[...] = D

    samp_iota = jax.lax.broadcasted_iota(jnp.int32, (npoint, nsample), 1)

    def topk_body(k, idx_acc):
        Dk = d_scratch[...]
        amin = jnp.argmin(Dk, axis=1).astype(jnp.int32)[:, None]
        idx_acc = jnp.where(samp_iota == k, amin, idx_acc)
        d_scratch[...] = jnp.where(lane_iota == amin, jnp.float32(jnp.inf), Dk)
        return idx_acc

    idx_ref[0] = jax.lax.fori_loop(
        0, nsample, topk_body, jnp.zeros((npoint, nsample), jnp.int32))


def _fps_knn(xyz_nc, npoint, nsample):
    B, N, _ = xyz_nc.shape
    xyz_cn = jnp.transpose(xyz_nc, (0, 2, 1))
    body = functools.partial(_fps_knn_body, npoint, nsample)
    new_xyz, idx = pl.pallas_call(
        body,
        grid=(B,),
        in_specs=[
            pl.BlockSpec((1, N, 3), lambda b: (b, 0, 0)),
            pl.BlockSpec((1, 3, N), lambda b: (b, 0, 0)),
        ],
        out_specs=[
            pl.BlockSpec((1, npoint, 3), lambda b: (b, 0, 0)),
            pl.BlockSpec((1, npoint, nsample), lambda b: (b, 0, 0)),
        ],
        out_shape=[
            jax.ShapeDtypeStruct((B, npoint, 3), jnp.float32),
            jax.ShapeDtypeStruct((B, npoint, nsample), jnp.int32),
        ],
        scratch_shapes=[pltpu.VMEM((npoint, N), jnp.float32)],
    )(xyz_nc, xyz_cn)
    return new_xyz, idx


def _compute_density(xyz, bandwidth):
    sq = _square_distance(xyz, xyz)
    g = jnp.exp(-sq / (2.0 * bandwidth * bandwidth)) / (2.5 * bandwidth)
    return jnp.mean(g, axis=-1)


def _density_body(bw, rows_ref, xyz_cn_ref, out_ref):
    rows = rows_ref[0]
    xyz_cn = xyz_cn_ref[0]
    D = -2.0 * jnp.dot(rows, xyz_cn)
    D = D + jnp.sum(rows ** 2, axis=1, keepdims=True)
    D = D + jnp.sum(xyz_cn ** 2, axis=0, keepdims=True)
    g = jnp.exp(-D / (2.0 * bw * bw)) / (2.5 * bw)
    out_ref[0] = 1.0 / jnp.mean(g, axis=1, keepdims=True)


def _inv_density(xyz_nc, bandwidth):
    B, N, _ = xyz_nc.shape
    rb = min(N, 512)
    xyz_cn = jnp.transpose(xyz_nc, (0, 2, 1))
    body = functools.partial(_density_body, bandwidth)
    out = pl.pallas_call(
        body,
        grid=(B, N // rb),
        in_specs=[
            pl.BlockSpec((1, rb, 3), lambda b, r: (b, r, 0)),
            pl.BlockSpec((1, 3, N), lambda b, r: (b, 0, 0)),
        ],
        out_specs=pl.BlockSpec((1, rb, 1), lambda b, r: (b, r, 0)),
        out_shape=jax.ShapeDtypeStruct((B, N, 1), jnp.float32),
    )(xyz_nc, xyz_cn)
    return out[:, :, 0]


def _conv_chain(layers, x):
    for L in layers:
        x = jnp.einsum('bckm,oc->bokm', x, L['w']) + L['b'][None, :, None, None]
        m = jnp.mean(x, axis=(0, 2, 3), keepdims=True)
        v = jnp.var(x, axis=(0, 2, 3), keepdims=True)
        x = (x - m) / jnp.sqrt(v + EPS) * L['g'][None, :, None, None] + L['beta'][None, :, None, None]
        x = jax.nn.relu(x)
    return x


def _pointconv_sa(p, xyz, points, npoint, nsample, bandwidth, group_all):
    B = xyz.shape[0]
    N = xyz.shape[2]
    xyz_t = jnp.transpose(xyz, (0, 2, 1))
    pts_t = jnp.transpose(points, (0, 2, 1))
    inv_density = _inv_density(xyz_t, bandwidth)
    if group_all:
        new_xyz = jnp.zeros((B, 1, 3), jnp.float32)
        grouped_xyz_norm = xyz_t[:, None, :, :]
        new_points = jnp.concatenate([grouped_xyz_norm, pts_t[:, None, :, :]], axis=-1)
        grouped_density = inv_density.reshape(B, 1, N, 1)
        npt = 1
    else:
        new_xyz, idx = _fps_knn(xyz_t, npoint, nsample)
        grouped_xyz = _index_points(xyz_t, idx)
        grouped_xyz_norm = grouped_xyz - new_xyz[:, :, None, :]
        grouped_points = _index_points(pts_t, idx)
        new_points = jnp.concatenate([grouped_xyz_norm, grouped_points], axis=-1)
        grouped_density = _index_points(inv_density[:, :, None], idx)
        npt = npoint
    x = jnp.transpose(new_points, (0, 3, 2, 1))
    x = _conv_chain(p['mlp'], x)
    inv_max = jnp.max(grouped_density, axis=2, keepdims=True)
    dscale = grouped_density / inv_max
    dscale = jnp.transpose(dscale, (0, 3, 2, 1))
    dscale = _conv_chain(p['dn'], dscale)
    x = x * dscale
    gx = jnp.transpose(grouped_xyz_norm, (0, 3, 2, 1))
    w = _conv_chain(p['wn'], gx)
    xp = jnp.transpose(x, (0, 3, 1, 2))
    wp = jnp.transpose(w, (0, 3, 2, 1))
    out = jnp.matmul(xp, wp).reshape(B, npt, -1)
    out = out @ p['lin_w'].T + p['lin_b']
    out = jnp.transpose(out, (0, 2, 1))
    m = jnp.mean(out, axis=(0, 2), keepdims=True)
    v = jnp.var(out, axis=(0, 2), keepdims=True)
    out = (out - m) / jnp.sqrt(v + EPS) * p['bnl_g'][None, :, None] + p['bnl_b'][None, :, None]
    out = jax.nn.relu(out)
    return jnp.transpose(new_xyz, (0, 2, 1)), out


_HI = jax.lax.Precision.HIGHEST


def _mk_stats(y, acc_ref, st_ref, i, last):
    @pl.when(i == 0)
    def _():
        acc_ref[...] = jnp.zeros_like(acc_ref)
    s = jnp.sum(y, axis=0, keepdims=True)
    ss = jnp.sum(y * y, axis=0, keepdims=True)
    acc_ref[...] = acc_ref[...] + jnp.concatenate([s, ss], axis=0)

    @pl.when(i == last)
    def _():
        st_ref[...] = acc_ref[...]


def _affine(st, g, beta, R):
    m = st[0] / R
    v = st[1] / R - m * m
    isr = g / jnp.sqrt(v + EPS)
    return isr, beta - m * isr


def _rows_op(x, wt=None, b=None, pre=None, mult=None, premult=None,
             k1row=None, rowsum=False, rb=8192):
    R, Cin = x.shape
    rb = min(rb, R)
    grid = R // rb
    if wt is not None:
        O = wt.shape[1]
    elif rowsum:
        O = 1
    elif k1row is not None:
        O = k1row.shape[1]
    else:
        O = Cin

    has_pre = pre is not None
    has_mult = mult is not None

    def body(*refs):
        refs = list(refs)
        x_ref = refs.pop(0)
        pre_ref = refs.pop(0) if has_pre else None
        mult_ref = refs.pop(0) if has_mult else None
        pm_ref = refs.pop(0) if has_mult else None
        w_ref = refs.pop(0) if (wt is not None or k1row is not None) else None
        b_ref = refs.pop(0) if b is not None else None
        y_ref, st_ref, acc_ref = refs
        i = pl.program_id(0)
        xv = x_ref[...]
        if has_pre:
            a = pre_ref[0:1]
            c = pre_ref[1:2]
            xv = jnp.maximum(xv * a + c, 0.0)
        if has_mult:
            mv = mult_ref[...]
            ma = pm_ref[0:1]
            mc = pm_ref[1:2]
            mv = jnp.maximum(mv * ma + mc, 0.0)
            xv = xv * mv
        if wt is not None:
            y = jnp.dot(xv, w_ref[...])
            if b is not None:
                y = y + b_ref[...]
        elif rowsum:
            y = jnp.sum(xv * w_ref[...], axis=1, keepdims=True)
            if b is not None:
                y = y + b_ref[0, 0]
        elif k1row is not None:
            y = xv * w_ref[...]
            if b is not None:
                y = y + b_ref[...]
        else:
            y = xv
        y_ref[...] = y
        _mk_stats(y, acc_ref, st_ref, i, grid - 1)

    in_arrays = [x]
    in_specs = [pl.BlockSpec((rb, Cin), lambda i: (i, 0))]
    if has_pre:
        in_arrays.append(jnp.concatenate([pre[0], pre[1]], axis=0))
        in_specs.append(pl.BlockSpec((2, Cin), lambda i: (0, 0)))
    if has_mult:
        in_arrays.append(mult)
        in_specs.append(pl.BlockSpec((rb, mult.shape[1]), lambda i: (i, 0)))
        in_arrays.append(jnp.concatenate([premult[0], premult[1]], axis=0))
        in_specs.append(pl.BlockSpec((2, mult.shape[1]), lambda i: (0, 0)))
    if wt is not None:
        in_arrays.append(wt)
        in_specs.append(pl.BlockSpec(wt.shape, lambda i: (0, 0)))
    elif k1row is not None:
        in_arrays.append(k1row)
        in_specs.append(pl.BlockSpec(k1row.shape, lambda i: (0, 0)))
    if b is not None:
        if rowsum:
            in_arrays.append(b.reshape(1, 1))
            in_specs.append(pl.BlockSpec(memory_space=pltpu.SMEM))
        else:
            in_arrays.append(b.reshape(1, -1))
            in_specs.append(pl.BlockSpec((1, O), lambda i: (0, 0)))

    y, st = pl.pallas_call(
        body,
        grid=(grid,),
        in_specs=in_specs,
        out_specs=[
            pl.BlockSpec((rb, O), lambda i: (i, 0)),
            pl.BlockSpec((2, O), lambda i: (0, 0)),
        ],
        out_shape=[
            jax.ShapeDtypeStruct((R, O), jnp.float32),
            jax.ShapeDtypeStruct((2, O), jnp.float32),
        ],
        scratch_shapes=[pltpu.VMEM((2, O), jnp.float32)],
    )(*in_arrays)
    return y, st


def _pool_lin(xr, prex, wr, prew, lw_r, lin_b, G, ns, gb=64):
    C = xr.shape[1]
    J = wr.shape[1]
    O = lw_r.shape[2]
    gb = min(gb, G)
    grid = G // gb

    def body(x_ref, px_ref, w_ref, pw_ref, lw_ref, lb_ref,
             y_ref, st_ref, acc_ref):
        i = pl.program_id(0)
        xv = x_ref[...]
        xv = jnp.maximum(xv * px_ref[0:1] + px_ref[1:2], 0.0)
        wv = w_ref[...]
        wv = jnp.maximum(wv * pw_ref[0:1] + pw_ref[1:2], 0.0)
        x3 = xv.reshape(gb, ns, C)
        w3 = wv.reshape(gb, ns, J)
        acc = jnp.zeros((gb, O), jnp.float32)
        for j in range(J):
            t = jnp.sum(x3 * w3[:, :, j:j + 1], axis=1)
            acc = acc + jnp.dot(t, lw_ref[j])
        y = acc + lb_ref[...]
        y_ref[...] = y
        _mk_stats(y, acc_ref, st_ref, i, grid - 1)

    return pl.pallas_call(
        body,
        grid=(grid,),
        in_specs=[
            pl.BlockSpec((gb * ns, C), lambda i: (i, 0)),
            pl.BlockSpec((2, C), lambda i: (0, 0)),
            pl.BlockSpec((gb * ns, J), lambda i: (i, 0)),
            pl.BlockSpec((2, J), lambda i: (0, 0)),
            pl.BlockSpec((J, C, O), lambda i: (0, 0, 0)),
            pl.BlockSpec((1, O), lambda i: (0, 0)),
        ],
        out_specs=[
            pl.BlockSpec((gb, O), lambda i: (i, 0)),
            pl.BlockSpec((2, O), lambda i: (0, 0)),
        ],
        out_shape=[
            jax.ShapeDtypeStruct((G, O), jnp.float32),
            jax.ShapeDtypeStruct((2, O), jnp.float32),
        ],
        scratch_shapes=[pltpu.VMEM((2, O), jnp.float32)],
    )(xr, jnp.concatenate([prex[0], prex[1]], 0), wr,
      jnp.concatenate([prew[0], prew[1]], 0), lw_r, lin_b.reshape(1, -1))


def _bn_apply(x, pre, relu=True, rb=8192):
    R, C = x.shape
    rb = min(rb, R)

    def body(x_ref, p_ref, y_ref):
        y = x_ref[...] * p_ref[0:1] + p_ref[1:2]
        y_ref[...] = jnp.maximum(y, 0.0) if relu else y

    return pl.pallas_call(
        body,
        grid=(R // rb,),
        in_specs=[
            pl.BlockSpec((rb, C), lambda i: (i, 0)),
            pl.BlockSpec((2, C), lambda i: (0, 0)),
        ],
        out_specs=pl.BlockSpec((rb, C), lambda i: (i, 0)),
        out_shape=jax.ShapeDtypeStruct((R, C), jnp.float32),
    )(x, jnp.concatenate([pre[0], pre[1]], 0))


def _gather_prep(table, idx, new_xyz, npoint, nsample, gb=64):
    B, N, C0 = table.shape
    ns = nsample
    grid_p = npoint // gb

    def body(idx_ref, tab_ref, nx_ref, feat_ref, gxn_ref, dsc_ref):
        rows = idx_ref[0]
        lane_iota = jax.lax.broadcasted_iota(jnp.int32, (1, N), 1)
        oh = (rows == lane_iota).astype(jnp.float32)
        gath = jnp.dot(oh, tab_ref[0], precision=_HI)
        grp_iota = jax.lax.broadcasted_iota(jnp.int32, (gb * ns, 1), 0) // ns
        gcol = jax.lax.broadcasted_iota(jnp.int32, (1, gb), 1)
        roh = (grp_iota == gcol).astype(jnp.float32)
        cent = jnp.dot(roh, nx_ref[0], precision=_HI)
        gxn = gath[:, :3] - cent
        gxn_ref[0] = gxn
        feat_ref[0] = jnp.concatenate([gxn, gath[:, 3:C0 - 1]], axis=1)
        gd = gath[:, C0 - 1:C0].reshape(gb, ns, 1)
        mx = jnp.max(gd, axis=1, keepdims=True)
        dsc_ref[0] = (gd / mx).reshape(gb * ns, 1)

    R = B * npoint * ns
    Cf = C0 - 4
    feats, gxn, dsc = pl.pallas_call(
        body,
        grid=(B, grid_p),
        in_specs=[
            pl.BlockSpec((1, gb * ns, 1), lambda b, p: (b, p, 0)),
            pl.BlockSpec((1, N, C0), lambda b, p: (b, 0, 0)),
            pl.BlockSpec((1, gb, 3), lambda b, p: (b, p, 0)),
        ],
        out_specs=[
            pl.BlockSpec((1, gb * ns, 3 + Cf), lambda b, p: (b * grid_p + p, 0, 0)),
            pl.BlockSpec((1, gb * ns, 3), lambda b, p: (b * grid_p + p, 0, 0)),
            pl.BlockSpec((1, gb * ns, 1), lambda b, p: (b * grid_p + p, 0, 0)),
        ],
        out_shape=[
            jax.ShapeDtypeStruct((B * grid_p, gb * ns, 3 + Cf), jnp.float32),
            jax.ShapeDtypeStruct((B * grid_p, gb * ns, 3), jnp.float32),
            jax.ShapeDtypeStruct((B * grid_p, gb * ns, 1), jnp.float32),
        ],
    )(idx.reshape(B, npoint * ns, 1), table, new_xyz)
    return (feats.reshape(R, 3 + Cf), gxn.reshape(R, 3), dsc.reshape(R, 1))


def _sa_dense(p, feats, gxn, dsc, G, ns):
    R = feats.shape[0]
    out_ch = p['lin_b'].shape[0]

    L = p['mlp'][0]
    y1, st1 = _rows_op(feats, wt=L['w'].T, b=L['b'])
    a1 = _affine(st1, L['g'].reshape(1, -1), L['beta'].reshape(1, -1), R)

    w1, w2, w3 = p['wn']
    z1, t1 = _rows_op(gxn, wt=w1['w'].T, b=w1['b'])
    pz1 = _affine(t1, w1['g'].reshape(1, -1), w1['beta'].reshape(1, -1), R)
    z2, t2 = _rows_op(z1, pre=pz1, wt=w2['w'].T, b=w2['b'])
    pz2 = _affine(t2, w2['g'].reshape(1, -1), w2['beta'].reshape(1, -1), R)
    z3, t3 = _rows_op(z2, pre=pz2, wt=w3['w'].T, b=w3['b'])
    pz3 = _affine(t3, w3['g'].reshape(1, -1), w3['beta'].reshape(1, -1), R)

    d1, u1 = p['dn'][0], p['dn'][1]
    dlast = p['dn'][2]
    e1, s1 = _rows_op(dsc, k1row=d1['w'].T, b=d1['b'])
    pe1 = _affine(s1, d1['g'].reshape(1, -1), d1['beta'].reshape(1, -1), R)
    e2, s2 = _rows_op(e1, pre=pe1, wt=u1['w'].T, b=u1['b'])
    pe2 = _affine(s2, u1['g'].reshape(1, -1), u1['beta'].reshape(1, -1), R)
    e3, s3 = _rows_op(e2, pre=pe2, k1row=dlast['w'], rowsum=True, b=dlast['b'])
    pe3 = _affine(s3, dlast['g'].reshape(1, -1), dlast['beta'].reshape(1, -1), R)

    xr, _ = _rows_op(y1, pre=a1, mult=e3, premult=pe3)
    J = 16
    C = y1.shape[1]
    lw_r = jnp.transpose(p['lin_w'].reshape(out_ch, C, J), (2, 1, 0))
    out, st = _pool_lin(xr, (jnp.zeros((1, C)) + 1.0, jnp.zeros((1, C))),
                        z3, pz3, lw_r, p['lin_b'], G, ns)
    pf = _affine(st, p['bnl_g'].reshape(1, -1), p['bnl_b'].reshape(1, -1), G)
    return _bn_apply(out, pf, relu=True)


def _ln_elu(x, g, beta):
    m = jnp.mean(x, axis=-1, keepdims=True)
    v = jnp.mean((x - m) ** 2, axis=-1, keepdims=True)
    x = (x - m) / jnp.sqrt(v + EPS) * g + beta
    return jnp.where(x > 0, x, jnp.exp(jnp.minimum(x, 0.0)) - 1.0)


def _qhead_body(q_ref, xpc_ref,
                w1q, b1q, g1q, beta1q,
                w2q, b2q, g2q, beta2q,
                w3q, b3q, g3q, beta3q,
                w1, b1, g1, beta1,
                w2, b2, g2, beta2,
                w3, b3,
                o_ref):
    q = q_ref[...]
    x = jnp.dot(q, w1q[...].T) + b1q[...]
    x = _ln_elu(x, g1q[...], beta1q[...])
    x = jnp.dot(x, w2q[...].T) + b2q[...]
    x = _ln_elu(x, g2q[...], beta2q[...])
    x = jnp.dot(x, w3q[...].T) + b3q[...]
    x = _ln_elu(x, g3q[...], beta3q[...])
    w1v = w1[...]
    wa = w1v[:, :256]
    wb = w1v[:, 256:]
    xpc = xpc_ref[0]
    cb = jnp.dot(xpc, wa.T)
    h = jnp.dot(x, wb.T) + cb + b1[...]
    h = _ln_elu(h, g1[...], beta1[...])
    h = jnp.dot(h, w2[...].T) + b2[...]
    h = _ln_elu(h, g2[...], beta2[...])
    o_ref[...] = jnp.sum(h * w3[...], axis=-1, keepdims=True) + b3[0, 0]


def _query_head(x_pc, query, params):
    B, num_qrs, _ = query.shape
    tok = B * num_qrs
    blk = 1024
    blocks_per_batch = num_qrs // blk
    q = query.reshape(tok, 3)

    def wspec(shape):
        return pl.BlockSpec(shape, lambda i: (0,) * len(shape))

    lins = []
    for name in ('fc1q', 'fc2q', 'fc3q', 'fc1', 'fc2'):
        L = params[name]
        lins += [L['w'], L['b'], L['g'], L['beta']]
    lins += [params['fc3']['w'], params['fc3']['b'].reshape(1, 1)]

    in_specs = [
        pl.BlockSpec((blk, 3), lambda i: (i, 0)),
        pl.BlockSpec((1, 1, 256), lambda i: (i // blocks_per_batch, 0, 0)),
    ] + [wspec(w.shape) for w in lins]
    in_specs[-1] = pl.BlockSpec(memory_space=pltpu.SMEM)

    out = pl.pallas_call(
        _qhead_body,
        grid=(tok // blk,),
        in_specs=in_specs,
        out_specs=pl.BlockSpec((blk, 1), lambda i: (i, 0)),
        out_shape=jax.ShapeDtypeStruct((tok, 1), jnp.float32),
    )(q, x_pc[:, None, :], *lins)
    return out


def _sa_stage(p, xyz_nc, pts, npoint, nsample, bandwidth):
    B, N, _ = xyz_nc.shape
    invd = _inv_density(xyz_nc, bandwidth)
    new_xyz, idx = _fps_knn(xyz_nc, npoint, nsample)
    table = jnp.concatenate([xyz_nc, pts, invd[:, :, None]], axis=-1)
    feats, gxn, dsc = _gather_prep(table, idx, new_xyz, npoint, nsample)
    out = _sa_dense(p, feats, gxn, dsc, B * npoint, nsample)
    return new_xyz, out


def kernel(pc, query, params):
    B = pc.shape[0]
    xyz0 = jnp.transpose(pc[:, :3, :], (0, 2, 1))
    pts0 = jnp.transpose(pc, (0, 2, 1))
    nx1, o1 = _sa_stage(params['sa1'], xyz0, pts0, 512, 32, 0.1)
    nx2, o2 = _sa_stage(params['sa2'], nx1, o1.reshape(B, 512, 64), 128, 64, 0.2)

    p3 = params['sa3']
    pts2 = o2.reshape(B, 128, 128)
    invd3 = _inv_density(nx2, 0.4)
    dsc3 = invd3 / jnp.max(invd3, axis=1, keepdims=True)
    feats3 = jnp.concatenate([nx2, pts2], axis=-1).reshape(B * 128, 131)
    gxn3 = nx2.reshape(B * 128, 3)
    x_pc = _sa_dense(p3, feats3, gxn3, dsc3.reshape(B * 128, 1), B, 128)
    return _query_head(x_pc, query, params)


# speedup vs baseline: 2.2864x; 1.0110x over previous
import functools

import jax
import jax.numpy as jnp
from jax.experimental import pallas as pl
from jax.experimental.pallas import tpu as pltpu

EPS = 1e-5


def _square_distance(src, dst):
    d = -2.0 * jnp.einsum('bnc,bmc->bnm', src, dst)
    d = d + jnp.sum(src ** 2, -1)[:, :, None]
    d = d + jnp.sum(dst ** 2, -1)[:, None, :]
    return d


def _index_points(points, idx):
    return jax.vmap(lambda p, i: p[i])(points, idx)


def _farthest_point_sample(xyz, npoint):
    B, N, _ = xyz.shape
    def body(i, state):
        cent, dist, far = state
        cent = cent.at[:, i].set(far)
        c = jnp.take_along_axis(xyz, far[:, None, None], axis=1)
        d = jnp.sum((xyz - c) ** 2, -1)
        dist = jnp.minimum(dist, d)
        far = jnp.argmax(dist, axis=-1).astype(jnp.int32)
        return cent, dist, far
    cent = jnp.zeros((B, npoint), jnp.int32)
    dist = jnp.full((B, N), 1e10, jnp.float32)
    far = jnp.zeros((B,), jnp.int32)
    cent, _, _ = jax.lax.fori_loop(0, npoint, body, (cent, dist, far))
    return cent


def _knn_point(nsample, xyz, new_xyz):
    d = _square_distance(new_xyz, xyz)
    _, idx = jax.lax.top_k(-d, nsample)
    return idx


def _fps_knn_body(npoint, nsample, xyz_nc_ref, xyz_cn_ref,
                  new_xyz_ref, idx_ref, d_scratch):
    xyz_nc = xyz_nc_ref[0]
    xyz_cn = xyz_cn_ref[0]
    N = xyz_cn.shape[1]
    lane_iota = jax.lax.broadcasted_iota(jnp.int32, (1, N), 1)
    cent_iota = jax.lax.broadcasted_iota(jnp.int32, (npoint, 1), 0)

    def body(i, carry):
        cent, dist, far = carry
        cent = jnp.where(cent_iota == i, far, cent)
        sel = lane_iota == far
        c = jnp.sum(jnp.where(sel, xyz_cn, 0.0), axis=1, keepdims=True)
        d = jnp.sum((xyz_cn - c) ** 2, axis=0, keepdims=True)
        dist = jnp.minimum(dist, d)
        far = jnp.argmax(dist).astype(jnp.int32)
        return cent, dist, far

    cent0 = jnp.zeros((npoint, 1), jnp.int32)
    dist0 = jnp.full((1, N), 1e10, jnp.float32)
    cent, _, _ = jax.lax.fori_loop(0, npoint, body,
                                   (cent0, dist0, jnp.int32(0)))

    onehot = (cent == lane_iota).astype(jnp.float32)
    new_xyz = jnp.dot(onehot, xyz_nc,
                      precision=jax.lax.Precision.HIGHEST)
    new_xyz_ref[0] = new_xyz

    D = -2.0 * jnp.dot(new_xyz, xyz_cn)
    D = D + jnp.sum(new_xyz ** 2, axis=1, keepdims=True)
    D = D + jnp.sum(xyz_cn ** 2, axis=0, keepdims=True)
    d_scratch[...] = D

    samp_iota = jax.lax.broadcasted_iota(jnp.int32, (npoint, nsample), 1)

    def topk_body(k, idx_acc):
        Dk = d_scratch[...]
        amin = jnp.argmin(Dk, axis=1).astype(jnp.int32)[:, None]
        idx_acc = jnp.where(samp_iota == k, amin, idx_acc)
        d_scratch[...] = jnp.where(lane_iota == amin, jnp.float32(jnp.inf), Dk)
        return idx_acc

    idx_ref[0] = jax.lax.fori_loop(
        0, nsample, topk_body, jnp.zeros((npoint, nsample), jnp.int32))


def _fps_knn(xyz_nc, npoint, nsample):
    B, N, _ = xyz_nc.shape
    xyz_cn = jnp.transpose(xyz_nc, (0, 2, 1))
    body = functools.partial(_fps_knn_body, npoint, nsample)
    new_xyz, idx = pl.pallas_call(
        body,
        grid=(B,),
        in_specs=[
            pl.BlockSpec((1, N, 3), lambda b: (b, 0, 0)),
            pl.BlockSpec((1, 3, N), lambda b: (b, 0, 0)),
        ],
        out_specs=[
            pl.BlockSpec((1, npoint, 3), lambda b: (b, 0, 0)),
            pl.BlockSpec((1, npoint, nsample), lambda b: (b, 0, 0)),
        ],
        out_shape=[
            jax.ShapeDtypeStruct((B, npoint, 3), jnp.float32),
            jax.ShapeDtypeStruct((B, npoint, nsample), jnp.int32),
        ],
        scratch_shapes=[pltpu.VMEM((npoint, N), jnp.float32)],
    )(xyz_nc, xyz_cn)
    return new_xyz, idx


def _compute_density(xyz, bandwidth):
    sq = _square_distance(xyz, xyz)
    g = jnp.exp(-sq / (2.0 * bandwidth * bandwidth)) / (2.5 * bandwidth)
    return jnp.mean(g, axis=-1)


def _density_body(bw, rows_ref, xyz_cn_ref, out_ref):
    rows = rows_ref[0]
    xyz_cn = xyz_cn_ref[0]
    D = -2.0 * jnp.dot(rows, xyz_cn)
    D = D + jnp.sum(rows ** 2, axis=1, keepdims=True)
    D = D + jnp.sum(xyz_cn ** 2, axis=0, keepdims=True)
    g = jnp.exp(-D / (2.0 * bw * bw)) / (2.5 * bw)
    out_ref[0] = 1.0 / jnp.mean(g, axis=1, keepdims=True)


def _inv_density(xyz_nc, bandwidth):
    B, N, _ = xyz_nc.shape
    rb = min(N, 512)
    xyz_cn = jnp.transpose(xyz_nc, (0, 2, 1))
    body = functools.partial(_density_body, bandwidth)
    out = pl.pallas_call(
        body,
        grid=(B, N // rb),
        in_specs=[
            pl.BlockSpec((1, rb, 3), lambda b, r: (b, r, 0)),
            pl.BlockSpec((1, 3, N), lambda b, r: (b, 0, 0)),
        ],
        out_specs=pl.BlockSpec((1, rb, 1), lambda b, r: (b, r, 0)),
        out_shape=jax.ShapeDtypeStruct((B, N, 1), jnp.float32),
    )(xyz_nc, xyz_cn)
    return out[:, :, 0]


def _conv_chain(layers, x):
    for L in layers:
        x = jnp.einsum('bckm,oc->bokm', x, L['w']) + L['b'][None, :, None, None]
        m = jnp.mean(x, axis=(0, 2, 3), keepdims=True)
        v = jnp.var(x, axis=(0, 2, 3), keepdims=True)
        x = (x - m) / jnp.sqrt(v + EPS) * L['g'][None, :, None, None] + L['beta'][None, :, None, None]
        x = jax.nn.relu(x)
    return x


def _pointconv_sa(p, xyz, points, npoint, nsample, bandwidth, group_all):
    B = xyz.shape[0]
    N = xyz.shape[2]
    xyz_t = jnp.transpose(xyz, (0, 2, 1))
    pts_t = jnp.transpose(points, (0, 2, 1))
    inv_density = _inv_density(xyz_t, bandwidth)
    if group_all:
        new_xyz = jnp.zeros((B, 1, 3), jnp.float32)
        grouped_xyz_norm = xyz_t[:, None, :, :]
        new_points = jnp.concatenate([grouped_xyz_norm, pts_t[:, None, :, :]], axis=-1)
        grouped_density = inv_density.reshape(B, 1, N, 1)
        npt = 1
    else:
        new_xyz, idx = _fps_knn(xyz_t, npoint, nsample)
        grouped_xyz = _index_points(xyz_t, idx)
        grouped_xyz_norm = grouped_xyz - new_xyz[:, :, None, :]
        grouped_points = _index_points(pts_t, idx)
        new_points = jnp.concatenate([grouped_xyz_norm, grouped_points], axis=-1)
        grouped_density = _index_points(inv_density[:, :, None], idx)
        npt = npoint
    x = jnp.transpose(new_points, (0, 3, 2, 1))
    x = _conv_chain(p['mlp'], x)
    inv_max = jnp.max(grouped_density, axis=2, keepdims=True)
    dscale = grouped_density / inv_max
    dscale = jnp.transpose(dscale, (0, 3, 2, 1))
    dscale = _conv_chain(p['dn'], dscale)
    x = x * dscale
    gx = jnp.transpose(grouped_xyz_norm, (0, 3, 2, 1))
    w = _conv_chain(p['wn'], gx)
    xp = jnp.transpose(x, (0, 3, 1, 2))
    wp = jnp.transpose(w, (0, 3, 2, 1))
    out = jnp.matmul(xp, wp).reshape(B, npt, -1)
    out = out @ p['lin_w'].T + p['lin_b']
    out = jnp.transpose(out, (0, 2, 1))
    m = jnp.mean(out, axis=(0, 2), keepdims=True)
    v = jnp.var(out, axis=(0, 2), keepdims=True)
    out = (out - m) / jnp.sqrt(v + EPS) * p['bnl_g'][None, :, None] + p['bnl_b'][None, :, None]
    out = jax.nn.relu(out)
    return jnp.transpose(new_xyz, (0, 2, 1)), out


_HI = jax.lax.Precision.HIGHEST


def _mk_stats(y, st_ref, nb):
    s = jnp.sum(y, axis=0, keepdims=True)
    mb = s / nb
    m2 = jnp.sum((y - mb) ** 2, axis=0, keepdims=True)
    st_ref[0] = jnp.concatenate([s, m2], axis=0)


def _affine(st, g, beta, R):
    nb = R / st.shape[0]
    s_b = st[:, 0]
    m2_b = st[:, 1]
    m = jnp.sum(s_b, axis=0, keepdims=True) / R
    mb = s_b / nb
    v = (jnp.sum(m2_b, axis=0, keepdims=True)
         + nb * jnp.sum((mb - m) ** 2, axis=0, keepdims=True)) / R
    isr = g / jnp.sqrt(v + EPS)
    return isr, beta - m * isr


def _rows_op(x, wt=None, b=None, pre=None, mult=None, premult=None,
             k1row=None, rowsum=False, rb=8192):
    R, Cin = x.shape
    rb = min(rb, R)
    grid = R // rb
    if wt is not None:
        O = wt.shape[1]
    elif rowsum:
        O = 1
    elif k1row is not None:
        O = k1row.shape[1]
    else:
        O = Cin

    has_pre = pre is not None
    has_mult = mult is not None

    def body(*refs):
        refs = list(refs)
        x_ref = refs.pop(0)
        pre_ref = refs.pop(0) if has_pre else None
        mult_ref = refs.pop(0) if has_mult else None
        pm_ref = refs.pop(0) if has_mult else None
        w_ref = refs.pop(0) if (wt is not None or k1row is not None) else None
        b_ref = refs.pop(0) if b is not None else None
        y_ref, st_ref = refs
        xv = x_ref[...]
        if has_pre:
            a = pre_ref[0:1]
            c = pre_ref[1:2]
            xv = jnp.maximum(xv * a + c, 0.0)
        if has_mult:
            mv = mult_ref[...]
            ma = pm_ref[0:1]
            mc = pm_ref[1:2]
            mv = jnp.maximum(mv * ma + mc, 0.0)
            xv = xv * mv
        if wt is not None:
            y = jnp.dot(xv, w_ref[...])
            if b is not None:
                y = y + b_ref[...]
        elif rowsum:
            y = jnp.sum(xv * w_ref[...], axis=1, keepdims=True)
            if b is not None:
                y = y + b_ref[0, 0]
        elif k1row is not None:
            y = xv * w_ref[...]
            if b is not None:
                y = y + b_ref[...]
        else:
            y = xv
        y_ref[...] = y
        _mk_stats(y, st_ref, rb)

    in_arrays = [x]
    in_specs = [pl.BlockSpec((rb, Cin), lambda i: (i, 0))]
    if has_pre:
        in_arrays.append(jnp.concatenate([pre[0], pre[1]], axis=0))
        in_specs.append(pl.BlockSpec((2, Cin), lambda i: (0, 0)))
    if has_mult:
        in_arrays.append(mult)
        in_specs.append(pl.BlockSpec((rb, mult.shape[1]), lambda i: (i, 0)))
        in_arrays.append(jnp.concatenate([premult[0], premult[1]], axis=0))
        in_specs.append(pl.BlockSpec((2, mult.shape[1]), lambda i: (0, 0)))
    if wt is not None:
        in_arrays.append(wt)
        in_specs.append(pl.BlockSpec(wt.shape, lambda i: (0, 0)))
    elif k1row is not None:
        in_arrays.append(k1row)
        in_specs.append(pl.BlockSpec(k1row.shape, lambda i: (0, 0)))
    if b is not None:
        if rowsum:
            in_arrays.append(b.reshape(1, 1))
            in_specs.append(pl.BlockSpec(memory_space=pltpu.SMEM))
        else:
            in_arrays.append(b.reshape(1, -1))
            in_specs.append(pl.BlockSpec((1, O), lambda i: (0, 0)))

    y, st = pl.pallas_call(
        body,
        grid=(grid,),
        in_specs=in_specs,
        out_specs=[
            pl.BlockSpec((rb, O), lambda i: (i, 0)),
            pl.BlockSpec((1, 2, O), lambda i: (i, 0, 0)),
        ],
        out_shape=[
            jax.ShapeDtypeStruct((R, O), jnp.float32),
            jax.ShapeDtypeStruct((grid, 2, O), jnp.float32),
        ],
    )(*in_arrays)
    return y, st


def _pool_lin(xr, prex, wr, prew, lw_r, lin_b, G, ns, gb=64):
    C = xr.shape[1]
    J = wr.shape[1]
    O = lw_r.shape[2]
    gb = min(gb, G)
    grid = G // gb

    def body(x_ref, px_ref, w_ref, pw_ref, lw_ref, lb_ref,
             y_ref, st_ref):
        xv = x_ref[...]
        xv = jnp.maximum(xv * px_ref[0:1] + px_ref[1:2], 0.0)
        wv = w_ref[...]
        wv = jnp.maximum(wv * pw_ref[0:1] + pw_ref[1:2], 0.0)
        x3 = xv.reshape(gb, ns, C)
        w3 = wv.reshape(gb, ns, J)
        acc = jnp.zeros((gb, O), jnp.float32)
        for j in range(J):
            t = jnp.sum(x3 * w3[:, :, j:j + 1], axis=1)
            acc = acc + jnp.dot(t, lw_ref[j])
        y = acc + lb_ref[...]
        y_ref[...] = y
        _mk_stats(y, st_ref, gb)

    return pl.pallas_call(
        body,
        grid=(grid,),
        in_specs=[
            pl.BlockSpec((gb * ns, C), lambda i: (i, 0)),
            pl.BlockSpec((2, C), lambda i: (0, 0)),
            pl.BlockSpec((gb * ns, J), lambda i: (i, 0)),
            pl.BlockSpec((2, J), lambda i: (0, 0)),
            pl.BlockSpec((J, C, O), lambda i: (0, 0, 0)),
            pl.BlockSpec((1, O), lambda i: (0, 0)),
        ],
        out_specs=[
            pl.BlockSpec((gb, O), lambda i: (i, 0)),
            pl.BlockSpec((1, 2, O), lambda i: (i, 0, 0)),
        ],
        out_shape=[
            jax.ShapeDtypeStruct((G, O), jnp.float32),
            jax.ShapeDtypeStruct((grid, 2, O), jnp.float32),
        ],
    )(xr, jnp.concatenate([prex[0], prex[1]], 0), wr,
      jnp.concatenate([prew[0], prew[1]], 0), lw_r, lin_b.reshape(1, -1))


def _bn_apply(x, pre, relu=True, rb=8192):
    R, C = x.shape
    rb = min(rb, R)

    def body(x_ref, p_ref, y_ref):
        y = x_ref[...] * p_ref[0:1] + p_ref[1:2]
        y_ref[...] = jnp.maximum(y, 0.0) if relu else y

    return pl.pallas_call(
        body,
        grid=(R // rb,),
        in_specs=[
            pl.BlockSpec((rb, C), lambda i: (i, 0)),
            pl.BlockSpec((2, C), lambda i: (0, 0)),
        ],
        out_specs=pl.BlockSpec((rb, C), lambda i: (i, 0)),
        out_shape=jax.ShapeDtypeStruct((R, C), jnp.float32),
    )(x, jnp.concatenate([pre[0], pre[1]], 0))


def _gather_prep(table, idx, new_xyz, npoint, nsample, gb=64):
    B, N, C0 = table.shape
    ns = nsample
    grid_p = npoint // gb

    def body(idx_ref, tab_ref, nx_ref, feat_ref, gxn_ref, dsc_ref):
        rows = idx_ref[0]
        lane_iota = jax.lax.broadcasted_iota(jnp.int32, (1, N), 1)
        oh = (rows == lane_iota).astype(jnp.float32)
        gath = jnp.dot(oh, tab_ref[0], precision=_HI)
        grp_iota = jax.lax.broadcasted_iota(jnp.int32, (gb * ns, 1), 0) // ns
        gcol = jax.lax.broadcasted_iota(jnp.int32, (1, gb), 1)
        roh = (grp_iota == gcol).astype(jnp.float32)
        cent = jnp.dot(roh, nx_ref[0], precision=_HI)
        gxn = gath[:, :3] - cent
        gxn_ref[0] = gxn
        feat_ref[0] = jnp.concatenate([gxn, gath[:, 3:C0 - 1]], axis=1)
        gd = gath[:, C0 - 1:C0].reshape(gb, ns, 1)
        mx = jnp.max(gd, axis=1, keepdims=True)
        dsc_ref[0] = (gd / mx).reshape(gb * ns, 1)

    R = B * npoint * ns
    Cf = C0 - 4
    feats, gxn, dsc = pl.pallas_call(
        body,
        grid=(B, grid_p),
        in_specs=[
            pl.BlockSpec((1, gb * ns, 1), lambda b, p: (b, p, 0)),
            pl.BlockSpec((1, N, C0), lambda b, p: (b, 0, 0)),
            pl.BlockSpec((1, gb, 3), lambda b, p: (b, p, 0)),
        ],
        out_specs=[
            pl.BlockSpec((1, gb * ns, 3 + Cf), lambda b, p: (b * grid_p + p, 0, 0)),
            pl.BlockSpec((1, gb * ns, 3), lambda b, p: (b * grid_p + p, 0, 0)),
            pl.BlockSpec((1, gb * ns, 1), lambda b, p: (b * grid_p + p, 0, 0)),
        ],
        out_shape=[
            jax.ShapeDtypeStruct((B * grid_p, gb * ns, 3 + Cf), jnp.float32),
            jax.ShapeDtypeStruct((B * grid_p, gb * ns, 3), jnp.float32),
            jax.ShapeDtypeStruct((B * grid_p, gb * ns, 1), jnp.float32),
        ],
    )(idx.reshape(B, npoint * ns, 1), table, new_xyz)
    return (feats.reshape(R, 3 + Cf), gxn.reshape(R, 3), dsc.reshape(R, 1))


def _sa_dense(p, feats, gxn, dsc, G, ns):
    R = feats.shape[0]
    out_ch = p['lin_b'].shape[0]

    L = p['mlp'][0]
    y1, st1 = _rows_op(feats, wt=L['w'].T, b=L['b'])
    a1 = _affine(st1, L['g'].reshape(1, -1), L['beta'].reshape(1, -1), R)

    w1, w2, w3 = p['wn']
    z1, t1 = _rows_op(gxn, wt=w1['w'].T, b=w1['b'])
    pz1 = _affine(t1, w1['g'].reshape(1, -1), w1['beta'].reshape(1, -1), R)
    z2, t2 = _rows_op(z1, pre=pz1, wt=w2['w'].T, b=w2['b'])
    pz2 = _affine(t2, w2['g'].reshape(1, -1), w2['beta'].reshape(1, -1), R)
    z3, t3 = _rows_op(z2, pre=pz2, wt=w3['w'].T, b=w3['b'])
    pz3 = _affine(t3, w3['g'].reshape(1, -1), w3['beta'].reshape(1, -1), R)

    d1, u1 = p['dn'][0], p['dn'][1]
    dlast = p['dn'][2]
    e1, s1 = _rows_op(dsc, k1row=d1['w'].T, b=d1['b'])
    pe1 = _affine(s1, d1['g'].reshape(1, -1), d1['beta'].reshape(1, -1), R)
    e2, s2 = _rows_op(e1, pre=pe1, wt=u1['w'].T, b=u1['b'])
    pe2 = _affine(s2, u1['g'].reshape(1, -1), u1['beta'].reshape(1, -1), R)
    e3, s3 = _rows_op(e2, pre=pe2, k1row=dlast['w'], rowsum=True, b=dlast['b'])
    pe3 = _affine(s3, dlast['g'].reshape(1, -1), dlast['beta'].reshape(1, -1), R)

    xr, _ = _rows_op(y1, pre=a1, mult=e3, premult=pe3)
    J = 16
    C = y1.shape[1]
    lw_r = jnp.transpose(p['lin_w'].reshape(out_ch, C, J), (2, 1, 0))
    out, st = _pool_lin(xr, (jnp.zeros((1, C)) + 1.0, jnp.zeros((1, C))),
                        z3, pz3, lw_r, p['lin_b'], G, ns)
    pf = _affine(st, p['bnl_g'].reshape(1, -1), p['bnl_b'].reshape(1, -1), G)
    return _bn_apply(out, pf, relu=True)


def _ln_elu(x, g, beta):
    m = jnp.mean(x, axis=-1, keepdims=True)
    v = jnp.mean((x - m) ** 2, axis=-1, keepdims=True)
    x = (x - m) / jnp.sqrt(v + EPS) * g + beta
    return jnp.where(x > 0, x, jnp.exp(jnp.minimum(x, 0.0)) - 1.0)


def _qhead_body(q_ref, xpc_ref,
                w1q, b1q, g1q, beta1q,
                w2q, b2q, g2q, beta2q,
                w3q, b3q, g3q, beta3q,
                w1, b1, g1, beta1,
                w2, b2, g2, beta2,
                w3, b3,
                o_ref):
    q = q_ref[...]
    x = jnp.dot(q, w1q[...].T) + b1q[...]
    x = _ln_elu(x, g1q[...], beta1q[...])
    x = jnp.dot(x, w2q[...].T) + b2q[...]
    x = _ln_elu(x, g2q[...], beta2q[...])
    x = jnp.dot(x, w3q[...].T) + b3q[...]
    x = _ln_elu(x, g3q[...], beta3q[...])
    w1v = w1[...]
    wa = w1v[:, :256]
    wb = w1v[:, 256:]
    xpc = xpc_ref[0]
    cb = jnp.dot(xpc, wa.T)
    h = jnp.dot(x, wb.T) + cb + b1[...]
    h = _ln_elu(h, g1[...], beta1[...])
    h = jnp.dot(h, w2[...].T) + b2[...]
    h = _ln_elu(h, g2[...], beta2[...])
    o_ref[...] = jnp.sum(h * w3[...], axis=-1, keepdims=True) + b3[0, 0]


def _query_head(x_pc, query, params):
    B, num_qrs, _ = query.shape
    tok = B * num_qrs
    blk = 1024
    blocks_per_batch = num_qrs // blk
    q = query.reshape(tok, 3)

    def wspec(shape):
        return pl.BlockSpec(shape, lambda i: (0,) * len(shape))

    lins = []
    for name in ('fc1q', 'fc2q', 'fc3q', 'fc1', 'fc2'):
        L = params[name]
        lins += [L['w'], L['b'], L['g'], L['beta']]
    lins += [params['fc3']['w'], params['fc3']['b'].reshape(1, 1)]

    in_specs = [
        pl.BlockSpec((blk, 3), lambda i: (i, 0)),
        pl.BlockSpec((1, 1, 256), lambda i: (i // blocks_per_batch, 0, 0)),
    ] + [wspec(w.shape) for w in lins]
    in_specs[-1] = pl.BlockSpec(memory_space=pltpu.SMEM)

    out = pl.pallas_call(
        _qhead_body,
        grid=(tok // blk,),
        in_specs=in_specs,
        out_specs=pl.BlockSpec((blk, 1), lambda i: (i, 0)),
        out_shape=jax.ShapeDtypeStruct((tok, 1), jnp.float32),
    )(q, x_pc[:, None, :], *lins)
    return out


def _sa_stage(p, xyz_nc, pts, npoint, nsample, bandwidth):
    B, N, _ = xyz_nc.shape
    invd = _inv_density(xyz_nc, bandwidth)
    new_xyz, idx = _fps_knn(xyz_nc, npoint, nsample)
    table = jnp.concatenate([xyz_nc, pts, invd[:, :, None]], axis=-1)
    feats, gxn, dsc = _gather_prep(table, idx, new_xyz, npoint, nsample)
    out = _sa_dense(p, feats, gxn, dsc, B * npoint, nsample)
    return new_xyz, out


def kernel(pc, query, params):
    B = pc.shape[0]
    xyz0 = jnp.transpose(pc[:, :3, :], (0, 2, 1))
    pts0 = jnp.transpose(pc, (0, 2, 1))
    nx1, o1 = _sa_stage(params['sa1'], xyz0, pts0, 512, 32, 0.1)
    nx2, o2 = _sa_stage(params['sa2'], nx1, o1.reshape(B, 512, 64), 128, 64, 0.2)

    p3 = params['sa3']
    pts2 = o2.reshape(B, 128, 128)
    invd3 = _inv_density(nx2, 0.4)
    dsc3 = invd3 / jnp.max(invd3, axis=1, keepdims=True)
    feats3 = jnp.concatenate([nx2, pts2], axis=-1).reshape(B * 128, 131)
    gxn3 = nx2.reshape(B * 128, 3)
    x_pc = _sa_dense(p3, feats3, gxn3, dsc3.reshape(B * 128, 1), B, 128)
    return _query_head(x_pc, query, params)
